# baseline (speedup 1.0000x reference)
import functools

import jax
import jax.numpy as jnp
from jax import lax
from jax.experimental import pallas as pl
from jax.experimental.pallas import tpu as pltpu
from jax.experimental.pallas import tpu_sc as plsc

_N = 10000
_E = 320000
_NC = 2
_NS = 16
_B = 112
_EPT = _E // _NS
_NCHUNKS = -(-_EPT // _B)
_EPT_PAD = _NCHUNKS * _B
_ROWS_PT = 624
_TAIL_OFF = _ROWS_PT * _NS
_TAIL = _N - _TAIL_OFF

_NBUF = 4
_LA = 2


def _make_spmm(D):
  hd = D // 2
  mesh = plsc.VectorSubcoreMesh(core_axis_name="c", subcore_axis_name="s")
  n_steps = (_NCHUNKS - _LA) // _NBUF
  n_loop = _NBUF * n_steps
  leftovers = list(range(n_loop, _NCHUNKS))

  @functools.partial(
      pl.kernel,
      out_type=jax.ShapeDtypeStruct((_N, D), jnp.float32),
      mesh=mesh,
      compiler_params=pltpu.CompilerParams(use_tc_tiling_on_sc=False),
      scratch_types=[
          pltpu.VMEM((_NCHUNKS, _B), jnp.int32),
          pltpu.VMEM((_NCHUNKS, _B), jnp.int32),
          pltpu.VMEM((_NCHUNKS, _B), jnp.float32),
          [pltpu.VMEM((_B, hd), jnp.float32)] * _NBUF,
          pltpu.VMEM_SHARED((_N, hd), jnp.float32),
          [pltpu.SemaphoreType.DMA] * _NBUF,
          [pltpu.SemaphoreType.DMA] * _NBUF,
          pltpu.SemaphoreType.DMA,
      ],
  )
  def spmm(x_hbm, src_hbm, dst_hbm, w_hbm, zeros_hbm, out_hbm,
           src_v, dst_v, w_v, rows, acc, g, sc, gi):
    c = lax.axis_index("c")
    s = lax.axis_index("s")
    ci0 = pltpu.async_copy(src_hbm.at[s], src_v, gi)
    ci1 = pltpu.async_copy(dst_hbm.at[s], dst_v, gi)
    ci2 = pltpu.async_copy(w_hbm.at[s], w_v, gi)
    pltpu.sync_copy(zeros_hbm.at[pl.ds(s * _ROWS_PT, _ROWS_PT)],
                    acc.at[pl.ds(s * _ROWS_PT, _ROWS_PT)])

    @pl.when(s == 0)
    def _zero_tail():
      pltpu.sync_copy(zeros_hbm.at[pl.ds(_TAIL_OFF, _TAIL)],
                      acc.at[pl.ds(_TAIL_OFF, _TAIL)])

    ci0.wait()
    ci1.wait()
    ci2.wait()
    cn = (c * _N).astype(jnp.int32)

    def add_off(i, carry):
      for d in range(_B // 16):
        sl = pl.ds(d * 16, 16)
        src_v[i, sl] = src_v[i, sl] + cn
      return carry

    lax.fori_loop(0, _NCHUNKS, add_off, 0)
    plsc.subcore_barrier()

    def gather_start(i, b):
      pltpu.async_copy(x_hbm.at[src_v.at[i]], rows[b], g[b])

    def gather_wait(b):
      pltpu.make_async_copy(x_hbm.at[src_v.at[0]], rows[b], g[b]).wait()

    def scat_start(i, b):
      pltpu.async_copy(rows[b], acc.at[dst_v.at[i]], sc[b], add=True)

    def scat_wait(b):
      pltpu.make_async_copy(rows[b], acc.at[dst_v.at[0]], sc[b]).wait()

    def scale(buf, i):
      @plsc.parallel_loop(0, _B // 16, 1, unroll=2)
      def grp(gg):
        wvec = w_v[i, pl.ds(gg * 16, 16)]
        for j in range(16):
          wj = wvec[j]
          k = gg * 16 + j
          for d in range(hd // 16):
            sl = pl.ds(d * 16, 16)
            buf[k, sl] = buf[k, sl] * wj

    for i in range(_LA):
      gather_start(i, i % _NBUF)

    def step(t, carry):
      for b in range(_NBUF):
        i = _NBUF * t + b
        nb = (b + _LA) % _NBUF

        @pl.when(i >= _NBUF - _LA)
        def _drain():
          scat_wait(nb)

        gather_start(i + _LA, nb)
        gather_wait(b)
        scale(rows[b], i)
        scat_start(i, b)
      return carry

    lax.fori_loop(0, n_steps, step, 0)
    for i in leftovers:
      b = i % _NBUF
      if i >= n_loop + _LA:
        scat_wait(b)
        gather_start(i, b)
      gather_wait(b)
      scale(rows[b], i)
      scat_start(i, b)
    for b in range(_NBUF):
      scat_wait(b)

    plsc.subcore_barrier()

    @pl.when(c == 0)
    def _write_lo():
      pltpu.sync_copy(acc.at[pl.ds(s * _ROWS_PT, _ROWS_PT)],
                      out_hbm.at[pl.ds(s * _ROWS_PT, _ROWS_PT), pl.ds(0, hd)])

      @pl.when(s == 0)
      def _write_lo_tail():
        pltpu.sync_copy(acc.at[pl.ds(_TAIL_OFF, _TAIL)],
                        out_hbm.at[pl.ds(_TAIL_OFF, _TAIL), pl.ds(0, hd)])

    @pl.when(c == 1)
    def _write_hi():
      pltpu.sync_copy(acc.at[pl.ds(s * _ROWS_PT, _ROWS_PT)],
                      out_hbm.at[pl.ds(s * _ROWS_PT, _ROWS_PT), pl.ds(hd, hd)])

      @pl.when(s == 0)
      def _write_hi_tail():
        pltpu.sync_copy(acc.at[pl.ds(_TAIL_OFF, _TAIL)],
                        out_hbm.at[pl.ds(_TAIL_OFF, _TAIL), pl.ds(hd, hd)])

  return spmm


_spmm_128 = _make_spmm(128)
_spmm_64 = _make_spmm(64)


def _tc_matmul_split(x, w):
  rows, cols = x.shape[0], w.shape[1]
  hd = cols // 2

  def body(x_ref, w_ref, o_ref):
    r = jnp.dot(x_ref[...], w_ref[...], preferred_element_type=jnp.float32)
    o_ref[0] = r[:, :hd]
    o_ref[1] = r[:, hd:]

  return pl.pallas_call(
      body,
      out_shape=jax.ShapeDtypeStruct((2, rows, hd), jnp.float32),
  )(x, w)


def _tc_relu_matmul_split(p, w):
  rows, cols = p.shape[0], w.shape[1]
  hd = cols // 2

  def body(p_ref, w_ref, o_ref):
    h = jnp.maximum(p_ref[...], 0.0)
    r = jnp.dot(h, w_ref[...], preferred_element_type=jnp.float32)
    o_ref[0] = r[:, :hd]
    o_ref[1] = r[:, hd:]

  return pl.pallas_call(
      body,
      out_shape=jax.ShapeDtypeStruct((2, rows, hd), jnp.float32),
  )(p, w)


def _prep(a):
  a = a.reshape(_NS, _EPT)
  a = jnp.pad(a, ((0, 0), (0, _EPT_PAD - _EPT)))
  return a.reshape(_NS, _NCHUNKS, _B)


def kernel(features, edge_index, edge_weight, W1, W2):
  src = _prep(edge_index[0].astype(jnp.int32))
  dst = _prep(edge_index[1].astype(jnp.int32))
  ew = _prep(edge_weight.astype(jnp.float32))
  z64 = jnp.zeros((_N, 64), jnp.float32)
  z32 = jnp.zeros((_N, 32), jnp.float32)

  xw = _tc_matmul_split(features, W1).reshape(2 * _N, 64)
  agg1 = _spmm_128(xw, src, dst, ew, z64)
  hw = _tc_relu_matmul_split(agg1, W2).reshape(2 * _N, 32)
  return _spmm_64(hw, src, dst, ew, z32)


# speedup vs baseline: 1.0258x; 1.0258x over previous
import functools

import jax
import jax.numpy as jnp
from jax import lax
from jax.experimental import pallas as pl
from jax.experimental.pallas import tpu as pltpu
from jax.experimental.pallas import tpu_sc as plsc

_N = 10000
_E = 320000
_NC = 2
_NS = 16
_B = 80
_EPT = _E // _NS
_NCHUNKS = -(-_EPT // _B)
_EPT_PAD = _NCHUNKS * _B
_ROWS_PT = 624
_TAIL_OFF = _ROWS_PT * _NS
_TAIL = _N - _TAIL_OFF

_NBUF = 4
_LA = 2


def _make_spmm(D):
  hd = D // 2
  mesh = plsc.VectorSubcoreMesh(core_axis_name="c", subcore_axis_name="s")
  n_steps = (_NCHUNKS - _LA) // _NBUF
  n_loop = _NBUF * n_steps
  leftovers = list(range(n_loop, _NCHUNKS))

  @functools.partial(
      pl.kernel,
      out_type=jax.ShapeDtypeStruct((_N, D), jnp.float32),
      mesh=mesh,
      compiler_params=pltpu.CompilerParams(use_tc_tiling_on_sc=False),
      scratch_types=[
          pltpu.VMEM((_NCHUNKS, _B), jnp.int32),
          pltpu.VMEM((_NCHUNKS, _B), jnp.int32),
          pltpu.VMEM((_NCHUNKS, _B), jnp.float32),
          [pltpu.VMEM((_B, hd), jnp.float32)] * _NBUF,
          pltpu.VMEM_SHARED((_N, hd), jnp.float32),
          [pltpu.SemaphoreType.DMA] * _NBUF,
          [pltpu.SemaphoreType.DMA] * _NBUF,
          pltpu.SemaphoreType.DMA,
      ],
  )
  def spmm(x_hbm, src_hbm, dst_hbm, w_hbm, zeros_hbm, out_hbm,
           src_v, dst_v, w_v, rows, acc, g, sc, gi):
    c = lax.axis_index("c")
    s = lax.axis_index("s")
    ci0 = pltpu.async_copy(src_hbm.at[c, s], src_v, gi)
    ci1 = pltpu.async_copy(dst_hbm.at[s], dst_v, gi)
    ci2 = pltpu.async_copy(w_hbm.at[s], w_v, gi)
    pltpu.sync_copy(zeros_hbm.at[pl.ds(s * _ROWS_PT, _ROWS_PT)],
                    acc.at[pl.ds(s * _ROWS_PT, _ROWS_PT)])

    @pl.when(s == 0)
    def _zero_tail():
      pltpu.sync_copy(zeros_hbm.at[pl.ds(_TAIL_OFF, _TAIL)],
                      acc.at[pl.ds(_TAIL_OFF, _TAIL)])

    ci0.wait()
    ci1.wait()
    ci2.wait()

    def gather_start(i, b):
      pltpu.async_copy(x_hbm.at[src_v.at[i]], rows[b], g[b])

    def gather_wait(b):
      pltpu.make_async_copy(x_hbm.at[src_v.at[0]], rows[b], g[b]).wait()

    def scat_start(i, b):
      pltpu.async_copy(rows[b], acc.at[dst_v.at[i]], sc[b], add=True)

    def scat_wait(b):
      pltpu.make_async_copy(rows[b], acc.at[dst_v.at[0]], sc[b]).wait()

    def scale(buf, i):
      @plsc.parallel_loop(0, _B // 16, 1, unroll=2)
      def grp(gg):
        wvec = w_v[i, pl.ds(gg * 16, 16)]
        for j in range(16):
          wj = wvec[j]
          k = gg * 16 + j
          for d in range(hd // 16):
            sl = pl.ds(d * 16, 16)
            buf[k, sl] = buf[k, sl] * wj

    for i in range(_LA):
      gather_start(i, i % _NBUF)
    plsc.subcore_barrier()

    def step(t, carry):
      for b in range(_NBUF):
        i = _NBUF * t + b
        nb = (b + _LA) % _NBUF

        @pl.when(i >= _NBUF - _LA)
        def _drain():
          scat_wait(nb)

        gather_start(i + _LA, nb)
        gather_wait(b)
        scale(rows[b], i)
        scat_start(i, b)
      return carry

    lax.fori_loop(0, n_steps, step, 0)
    for i in leftovers:
      b = i % _NBUF
      if i >= n_loop + _LA:
        scat_wait(b)
        gather_start(i, b)
      gather_wait(b)
      scale(rows[b], i)
      scat_start(i, b)
    for b in range(_NBUF):
      scat_wait(b)

    plsc.subcore_barrier()

    @pl.when(c == 0)
    def _write_lo():
      pltpu.sync_copy(acc.at[pl.ds(s * _ROWS_PT, _ROWS_PT)],
                      out_hbm.at[pl.ds(s * _ROWS_PT, _ROWS_PT), pl.ds(0, hd)])

      @pl.when(s == 0)
      def _write_lo_tail():
        pltpu.sync_copy(acc.at[pl.ds(_TAIL_OFF, _TAIL)],
                        out_hbm.at[pl.ds(_TAIL_OFF, _TAIL), pl.ds(0, hd)])

    @pl.when(c == 1)
    def _write_hi():
      pltpu.sync_copy(acc.at[pl.ds(s * _ROWS_PT, _ROWS_PT)],
                      out_hbm.at[pl.ds(s * _ROWS_PT, _ROWS_PT), pl.ds(hd, hd)])

      @pl.when(s == 0)
      def _write_hi_tail():
        pltpu.sync_copy(acc.at[pl.ds(_TAIL_OFF, _TAIL)],
                        out_hbm.at[pl.ds(_TAIL_OFF, _TAIL), pl.ds(hd, hd)])

  return spmm


_spmm_128 = _make_spmm(128)
_spmm_64 = _make_spmm(64)


def _tc_matmul_split(x, w):
  rows, cols = x.shape[0], w.shape[1]
  hd = cols // 2

  def body(x_ref, w_ref, o_ref):
    r = jnp.dot(x_ref[...], w_ref[...], preferred_element_type=jnp.float32)
    o_ref[0] = r[:, :hd]
    o_ref[1] = r[:, hd:]

  return pl.pallas_call(
      body,
      out_shape=jax.ShapeDtypeStruct((2, rows, hd), jnp.float32),
  )(x, w)


def _tc_relu_matmul_split(p, w):
  rows, cols = p.shape[0], w.shape[1]
  hd = cols // 2

  def body(p_ref, w_ref, o_ref):
    h = jnp.maximum(p_ref[...], 0.0)
    r = jnp.dot(h, w_ref[...], preferred_element_type=jnp.float32)
    o_ref[0] = r[:, :hd]
    o_ref[1] = r[:, hd:]

  return pl.pallas_call(
      body,
      out_shape=jax.ShapeDtypeStruct((2, rows, hd), jnp.float32),
  )(p, w)


def _prep(a):
  a = a.reshape(_NS, _EPT)
  a = jnp.pad(a, ((0, 0), (0, _EPT_PAD - _EPT)))
  return a.reshape(_NS, _NCHUNKS, _B)


def kernel(features, edge_index, edge_weight, W1, W2):
  s3 = _prep(edge_index[0].astype(jnp.int32))
  src = jnp.stack([s3, s3 + _N])
  dst = _prep(edge_index[1].astype(jnp.int32))
  ew = _prep(edge_weight.astype(jnp.float32))
  z64 = jnp.zeros((_N, 64), jnp.float32)
  z32 = jnp.zeros((_N, 32), jnp.float32)

  xw = _tc_matmul_split(features, W1).reshape(2 * _N, 64)
  agg1 = _spmm_128(xw, src, dst, ew, z64)
  hw = _tc_relu_matmul_split(agg1, W2).reshape(2 * _N, 32)
  return _spmm_64(hw, src, dst, ew, z32)


# speedup vs baseline: 1.0510x; 1.0246x over previous
import functools

import jax
import jax.numpy as jnp
from jax import lax
from jax.experimental import pallas as pl
from jax.experimental.pallas import tpu as pltpu
from jax.experimental.pallas import tpu_sc as plsc

_N = 10000
_E = 320000
_NC = 2
_NS = 16
_B = 80
_EPT = _E // _NS
_NCHUNKS = -(-_EPT // _B)
_EPT_PAD = _NCHUNKS * _B
_ROWS_PT = 624
_TAIL_OFF = _ROWS_PT * _NS
_TAIL = _N - _TAIL_OFF

_NBUF = 4
_LA = 2


def _make_spmm(D):
  hd = D // 2
  mesh = plsc.VectorSubcoreMesh(core_axis_name="c", subcore_axis_name="s")
  n_steps = (_NCHUNKS - _LA) // _NBUF
  n_loop = _NBUF * n_steps
  leftovers = list(range(n_loop, _NCHUNKS))

  @functools.partial(
      pl.kernel,
      out_type=jax.ShapeDtypeStruct((_N, D), jnp.float32),
      mesh=mesh,
      compiler_params=pltpu.CompilerParams(use_tc_tiling_on_sc=False),
      scratch_types=[
          pltpu.VMEM((_NCHUNKS, _B), jnp.int32),
          pltpu.VMEM((_NCHUNKS, _B), jnp.int32),
          pltpu.VMEM((_NCHUNKS, _B), jnp.float32),
          [pltpu.VMEM((_B, hd), jnp.float32)] * _NBUF,
          pltpu.VMEM_SHARED((_N, hd), jnp.float32),
          [pltpu.SemaphoreType.DMA] * _NBUF,
          [pltpu.SemaphoreType.DMA] * _NBUF,
          pltpu.SemaphoreType.DMA,
      ],
  )
  def spmm(x_hbm, src_hbm, dst_hbm, w_hbm, zeros_hbm, out_hbm,
           src_v, dst_v, w_v, rows, acc, g, sc, gi):
    c = lax.axis_index("c")
    s = lax.axis_index("s")
    ci0 = pltpu.async_copy(src_hbm.at[s], src_v, gi)
    ci1 = pltpu.async_copy(dst_hbm.at[s], dst_v, gi)
    ci2 = pltpu.async_copy(w_hbm.at[s], w_v, gi)
    pltpu.sync_copy(zeros_hbm.at[pl.ds(s * _ROWS_PT, _ROWS_PT)],
                    acc.at[pl.ds(s * _ROWS_PT, _ROWS_PT)])

    @pl.when(s == 0)
    def _zero_tail():
      pltpu.sync_copy(zeros_hbm.at[pl.ds(_TAIL_OFF, _TAIL)],
                      acc.at[pl.ds(_TAIL_OFF, _TAIL)])

    ci0.wait()
    ci1.wait()
    ci2.wait()
    cn = (c * _N).astype(jnp.int32)

    def add_off(i, carry):
      for d in range(_B // 16):
        sl = pl.ds(d * 16, 16)
        src_v[i, sl] = src_v[i, sl] + cn
      return carry

    lax.fori_loop(0, _NCHUNKS, add_off, 0)

    def gather_start(i, b):
      pltpu.async_copy(x_hbm.at[src_v.at[i]], rows[b], g[b])

    def gather_wait(b):
      pltpu.make_async_copy(x_hbm.at[src_v.at[0]], rows[b], g[b]).wait()

    def scat_start(i, b):
      pltpu.async_copy(rows[b], acc.at[dst_v.at[i]], sc[b], add=True)

    def scat_wait(b):
      pltpu.make_async_copy(rows[b], acc.at[dst_v.at[0]], sc[b]).wait()

    def scale(buf, i):
      @plsc.parallel_loop(0, _B // 16, 1, unroll=2)
      def grp(gg):
        wvec = w_v[i, pl.ds(gg * 16, 16)]
        for j in range(16):
          wj = wvec[j]
          k = gg * 16 + j
          for d in range(hd // 16):
            sl = pl.ds(d * 16, 16)
            buf[k, sl] = buf[k, sl] * wj

    for i in range(_LA):
      gather_start(i, i % _NBUF)
    plsc.subcore_barrier()

    def step(t, carry):
      for b in range(_NBUF):
        i = _NBUF * t + b
        nb = (b + _LA) % _NBUF

        @pl.when(i >= _NBUF - _LA)
        def _drain():
          scat_wait(nb)

        gather_start(i + _LA, nb)
        gather_wait(b)
        scale(rows[b], i)
        scat_start(i, b)
      return carry

    lax.fori_loop(0, n_steps, step, 0)
    for i in leftovers:
      b = i % _NBUF
      if i >= n_loop + _LA:
        scat_wait(b)
        gather_start(i, b)
      gather_wait(b)
      scale(rows[b], i)
      scat_start(i, b)
    for b in range(_NBUF):
      scat_wait(b)

    plsc.subcore_barrier()

    @pl.when(c == 0)
    def _write_lo():
      pltpu.sync_copy(acc.at[pl.ds(s * _ROWS_PT, _ROWS_PT)],
                      out_hbm.at[pl.ds(s * _ROWS_PT, _ROWS_PT), pl.ds(0, hd)])

      @pl.when(s == 0)
      def _write_lo_tail():
        pltpu.sync_copy(acc.at[pl.ds(_TAIL_OFF, _TAIL)],
                        out_hbm.at[pl.ds(_TAIL_OFF, _TAIL), pl.ds(0, hd)])

    @pl.when(c == 1)
    def _write_hi():
      pltpu.sync_copy(acc.at[pl.ds(s * _ROWS_PT, _ROWS_PT)],
                      out_hbm.at[pl.ds(s * _ROWS_PT, _ROWS_PT), pl.ds(hd, hd)])

      @pl.when(s == 0)
      def _write_hi_tail():
        pltpu.sync_copy(acc.at[pl.ds(_TAIL_OFF, _TAIL)],
                        out_hbm.at[pl.ds(_TAIL_OFF, _TAIL), pl.ds(hd, hd)])

  return spmm


_spmm_128 = _make_spmm(128)
_spmm_64 = _make_spmm(64)


def _tc_matmul_split(x, w):
  rows, cols = x.shape[0], w.shape[1]
  hd = cols // 2

  def body(x_ref, w_ref, o_ref):
    r = jnp.dot(x_ref[...], w_ref[...], preferred_element_type=jnp.float32)
    o_ref[0] = r[:, :hd]
    o_ref[1] = r[:, hd:]

  return pl.pallas_call(
      body,
      out_shape=jax.ShapeDtypeStruct((2, rows, hd), jnp.float32),
  )(x, w)


def _tc_relu_matmul_split(p, w):
  rows, cols = p.shape[0], w.shape[1]
  hd = cols // 2

  def body(p_ref, w_ref, o_ref):
    h = jnp.maximum(p_ref[...], 0.0)
    r = jnp.dot(h, w_ref[...], preferred_element_type=jnp.float32)
    o_ref[0] = r[:, :hd]
    o_ref[1] = r[:, hd:]

  return pl.pallas_call(
      body,
      out_shape=jax.ShapeDtypeStruct((2, rows, hd), jnp.float32),
  )(p, w)


def _prep(a):
  a = a.reshape(_NS, _EPT)
  a = jnp.pad(a, ((0, 0), (0, _EPT_PAD - _EPT)))
  return a.reshape(_NS, _NCHUNKS, _B)


def kernel(features, edge_index, edge_weight, W1, W2):
  src = _prep(edge_index[0].astype(jnp.int32))
  dst = _prep(edge_index[1].astype(jnp.int32))
  ew = _prep(edge_weight.astype(jnp.float32))
  z64 = jnp.zeros((_N, 64), jnp.float32)
  z32 = jnp.zeros((_N, 32), jnp.float32)

  xw = _tc_matmul_split(features, W1).reshape(2 * _N, 64)
  agg1 = _spmm_128(xw, src, dst, ew, z64)
  hw = _tc_relu_matmul_split(agg1, W2).reshape(2 * _N, 32)
  return _spmm_64(hw, src, dst, ew, z32)
